# baseline (speedup 1.0000x reference)
import functools

import jax
import jax.numpy as jnp
from jax import lax
from jax.experimental import pallas as pl
from jax.experimental.pallas import tpu as pltpu
from jax.experimental.pallas import tpu_sc as plsc

NC = 2
NS = 16
CH = 128


def _sc_agg(table, srci, dst2, z128, n, ep, split):
    ept = ep // (NC * NS) if split else ep // NS
    ncz = ept // CH
    gr = NS * 8
    n_sink = ((n + gr - 1) // gr) * gr
    rpt = n_sink // NS

    mesh = plsc.VectorSubcoreMesh(core_axis_name="c", subcore_axis_name="s",
                                  num_cores=NC, num_subcores=NS)
    ib = 16
    assert ncz % ib == 0

    @functools.partial(
        pl.kernel, mesh=mesh,
        out_type=jax.ShapeDtypeStruct((2 * n_sink, 128), jnp.float32),
        scratch_types=[
            pltpu.VMEM_SHARED((n_sink, 128), jnp.float32),
            pltpu.VMEM((ib, CH), jnp.int32),
            pltpu.VMEM((ib, CH), jnp.int32),
            pltpu.VMEM((CH, 128), jnp.float32),
            pltpu.VMEM((CH, 128), jnp.float32),
            pltpu.SemaphoreType.DMA,
            pltpu.SemaphoreType.DMA,
            pltpu.SemaphoreType.DMA,
        ],
    )
    def k(tab_hbm, src_hbm, dst_hbm, z128_hbm, agg_out,
          sp_agg, vm_src, vm_dst, vm_rows0, vm_rows1, sem0, sem1, isem):
        c = lax.axis_index("c")
        s = lax.axis_index("s")
        if split:
            sbase = pl.multiple_of((c * NS + s) * ncz, ncz)
            rbase = sbase
        else:
            sbase = pl.multiple_of(c * (ep // CH) + s * ncz, ncz)
            rbase = pl.multiple_of(s * ncz, ncz)
        pltpu.sync_copy(z128_hbm, vm_rows0)
        zb = pl.multiple_of(s * rpt, 8)
        done = 0
        while done < rpt:
            step = min(128, rpt - done)
            pltpu.sync_copy(vm_rows0.at[pl.ds(0, step)],
                            sp_agg.at[pl.ds(zb + done, step)])
            done += step
        plsc.subcore_barrier()

        bufs = (vm_rows0, vm_rows1)
        sems = (sem0, sem1)

        def body(jo, carry):
            i0 = pltpu.async_copy(
                src_hbm.at[pl.ds(pl.multiple_of(sbase + jo * ib, ib), ib)],
                vm_src, isem)
            i1 = pltpu.async_copy(
                dst_hbm.at[pl.ds(pl.multiple_of(rbase + jo * ib, ib), ib)],
                vm_dst, isem)
            i0.wait()
            i1.wait()
            nq = 4
            qs = CH // nq

            def gather_chunk(b):
                buf = bufs[b % 2]
                return [
                    pltpu.async_copy(
                        tab_hbm.at[vm_src.at[b, pl.ds(q * qs, qs)]],
                        buf.at[pl.ds(q * qs, qs)], sems[b % 2])
                    for q in range(nq)
                ]

            gd = [None, None]
            gd[0] = gather_chunk(0)
            for b in range(ib):
                if b + 1 < ib:
                    gd[(b + 1) % 2] = gather_chunk(b + 1)
                for d in gd[b % 2]:
                    d.wait()
                pltpu.sync_copy(bufs[b % 2], sp_agg.at[vm_dst.at[b]], add=True)
            return carry

        lax.fori_loop(0, ncz // ib, body, 0)
        plsc.subcore_barrier()

        ob = pl.multiple_of(s * rpt, 8)
        obo = pl.multiple_of(c * n_sink + s * rpt, 8)
        done = 0
        while done < rpt:
            step = min(128, rpt - done)
            pltpu.sync_copy(sp_agg.at[pl.ds(ob + done, step)], vm_rows0.at[pl.ds(0, step)])
            pltpu.sync_copy(vm_rows0.at[pl.ds(0, step)],
                            agg_out.at[pl.ds(pl.multiple_of(obo + done, 8), step)])
            done += step

    return k(table, srci, dst2, z128)


def _sc_deg(dst2, z128, ones128, n, ep):
    ept = ep // (NC * NS)
    ncz = ept // CH
    gr = NS * 8
    n_sink = ((n + gr - 1) // gr) * gr
    rpt = n_sink // NS

    mesh = plsc.VectorSubcoreMesh(core_axis_name="c", subcore_axis_name="s",
                                  num_cores=NC, num_subcores=NS)
    ib = 8
    assert ncz % ib == 0

    @functools.partial(
        pl.kernel, mesh=mesh,
        out_type=jax.ShapeDtypeStruct((2 * n_sink, 128), jnp.float32),
        scratch_types=[
            pltpu.VMEM_SHARED((n_sink, 128), jnp.float32),
            pltpu.VMEM((ib, CH), jnp.int32),
            pltpu.VMEM((128, 128), jnp.float32),
            pltpu.VMEM((CH, 128), jnp.float32),
        ],
    )
    def k(dst_hbm, z128_hbm, ones_hbm, deg_out, sp_deg, vm_dst, vm_z, vm_ones):
        c = lax.axis_index("c")
        s = lax.axis_index("s")
        rbase = pl.multiple_of((c * NS + s) * ncz, ncz)
        pltpu.sync_copy(z128_hbm, vm_z)
        pltpu.sync_copy(ones_hbm, vm_ones)
        zb = pl.multiple_of(s * rpt, 8)
        done = 0
        while done < rpt:
            step = min(128, rpt - done)
            pltpu.sync_copy(vm_z.at[pl.ds(0, step)],
                            sp_deg.at[pl.ds(zb + done, step)])
            done += step
        plsc.subcore_barrier()

        def body(jo, carry):
            pltpu.sync_copy(
                dst_hbm.at[pl.ds(pl.multiple_of(rbase + jo * ib, ib), ib)], vm_dst)
            for b in range(ib):
                pltpu.sync_copy(vm_ones, sp_deg.at[vm_dst.at[b]], add=True)
            return carry

        lax.fori_loop(0, ncz // ib, body, 0)
        plsc.subcore_barrier()

        ob = pl.multiple_of(s * rpt, 8)
        obo = pl.multiple_of(c * n_sink + s * rpt, 8)
        done = 0
        while done < rpt:
            step = min(128, rpt - done)
            pltpu.sync_copy(sp_deg.at[pl.ds(ob + done, step)], vm_z.at[pl.ds(0, step)])
            pltpu.sync_copy(vm_z.at[pl.ds(0, step)],
                            deg_out.at[pl.ds(pl.multiple_of(obo + done, 8), step)])
            done += step

    return k(dst2, z128, ones128)


def _ln_prelu(pre, g, b, al):
    mu = jnp.mean(pre, axis=-1, keepdims=True)
    var = jnp.mean((pre - mu) ** 2, axis=-1, keepdims=True)
    h = (pre - mu) * jax.lax.rsqrt(var + 1e-5) * g + b
    return jnp.where(h >= 0, h, al * h)


def _dot(a, b):
    return jnp.dot(a, b, preferred_element_type=jnp.float32)


def _row_specs(bn, shapes):
    return [pl.BlockSpec((bn,) + tuple(s[1:]),
                         lambda i, r=len(s) - 1: (i,) + (0,) * r)
            for s in shapes]


def _full_specs(shapes):
    return [pl.BlockSpec(tuple(s), lambda i, r=len(s): (0,) * r)
            for s in shapes]


def _t1(x, a0, a1, dg0, dg1, WsT, WnT, WskT, b, g, be, al, n, bn):
    def body(x_r, a0_r, a1_r, dg0_r, dg1_r, WsT_r, WnT_r, WskT_r,
             b_r, g_r, be_r, al_r, h1_r, in2_r):
        x_ = x_r[...]
        agg = a0_r[...] + a1_r[...]
        d = jnp.maximum(dg0_r[:, 0:1] + dg1_r[:, 0:1], 1.0)
        pre = _dot(x_, WsT_r[...]) + _dot(agg, WnT_r[...]) / d + b_r[...]
        h1 = _ln_prelu(pre, g_r[...], be_r[...], al_r[0, 0])
        h1_r[...] = h1
        in2_r[...] = h1 + _dot(x_, WskT_r[...])

    dh = WsT.shape[1]
    return pl.pallas_call(
        body,
        grid=(n // bn,),
        in_specs=_row_specs(bn, [x.shape, a0.shape, a1.shape, dg0.shape, dg1.shape])
        + _full_specs([WsT.shape, WnT.shape, WskT.shape, b.shape, g.shape,
                       be.shape, al.shape]),
        out_specs=_row_specs(bn, [(n, dh), (n, dh)]),
        out_shape=[jax.ShapeDtypeStruct((n, dh), jnp.float32),
                   jax.ShapeDtypeStruct((n, dh), jnp.float32)],
    )(x, a0, a1, dg0, dg1, WsT, WnT, WskT, b, g, be, al)


def _t2(x, in2, ah0, ah1, dg0, dg1, h1, WsT, WnAT, WnBT, WskT, b, g, be, al,
        n, bn):
    def body(x_r, in2_r, ah0_r, ah1_r, dg0_r, dg1_r, h1_r,
             WsT_r, WnAT_r, WnBT_r, WskT_r, b_r, g_r, be_r, al_r, in3_r):
        d = jnp.maximum(dg0_r[:, 0:1] + dg1_r[:, 0:1], 1.0)
        neigh = (_dot(ah0_r[...], WnAT_r[...]) + _dot(ah1_r[...], WnBT_r[...])) / d
        pre = _dot(in2_r[...], WsT_r[...]) + neigh + b_r[...]
        h2 = _ln_prelu(pre, g_r[...], be_r[...], al_r[0, 0])
        in3_r[...] = h1_r[...] + h2 + _dot(x_r[...], WskT_r[...])

    dh = WsT.shape[1]
    return pl.pallas_call(
        body,
        grid=(n // bn,),
        in_specs=_row_specs(bn, [x.shape, in2.shape, ah0.shape, ah1.shape,
                                 dg0.shape, dg1.shape, h1.shape])
        + _full_specs([WsT.shape, WnAT.shape, WnBT.shape, WskT.shape,
                       b.shape, g.shape, be.shape, al.shape]),
        out_specs=_row_specs(bn, [(n, dh)])[0],
        out_shape=jax.ShapeDtypeStruct((n, dh), jnp.float32),
    )(x, in2, ah0, ah1, dg0, dg1, h1, WsT, WnAT, WnBT, WskT, b, g, be, al)


def _t3(in3, ah0, ah1, dg0, dg1, WsT, WnAT, WnBT, b, g, be, al, n, bn):
    def body(in3_r, ah0_r, ah1_r, dg0_r, dg1_r,
             WsT_r, WnAT_r, WnBT_r, b_r, g_r, be_r, al_r, out_r):
        d = jnp.maximum(dg0_r[:, 0:1] + dg1_r[:, 0:1], 1.0)
        neigh = (_dot(ah0_r[...], WnAT_r[...]) + _dot(ah1_r[...], WnBT_r[...])) / d
        pre = _dot(in3_r[...], WsT_r[...]) + neigh + b_r[...]
        out_r[...] = _ln_prelu(pre, g_r[...], be_r[...], al_r[0, 0])

    dh = WsT.shape[1]
    return pl.pallas_call(
        body,
        grid=(n // bn,),
        in_specs=_row_specs(bn, [in3.shape, ah0.shape, ah1.shape,
                                 dg0.shape, dg1.shape])
        + _full_specs([WsT.shape, WnAT.shape, WnBT.shape, b.shape, g.shape,
                       be.shape, al.shape]),
        out_specs=_row_specs(bn, [(n, dh)])[0],
        out_shape=jax.ShapeDtypeStruct((n, dh), jnp.float32),
    )(in3, ah0, ah1, dg0, dg1, WsT, WnAT, WnBT, b, g, be, al)


def kernel(x, edge_index, W_self0, W_neigh0, b0, W_self1, W_neigh1, b1,
           W_self2, W_neigh2, b2, W_skip0, W_skip1,
           ln_g0, ln_b0, ln_g1, ln_b1, ln_g2, ln_b2,
           alpha0, alpha1, alpha2):
    n, d_in = x.shape
    dh = W_self0.shape[0]
    e = edge_index.shape[1]
    bn = 2000

    gran = NC * NS * CH * 8
    ep = ((e + gran - 1) // gran) * gran
    nsk = ((n + NS * 8 - 1) // (NS * 8)) * (NS * 8)
    pad = ep - e
    src = edge_index[0]
    dst = edge_index[1]
    src_p = jnp.concatenate([src, jnp.zeros((pad,), jnp.int32)])
    dst_p = jnp.concatenate([dst, jnp.full((pad,), n, jnp.int32)])
    dst2 = dst_p.reshape(ep // CH, CH)
    src1 = src_p.reshape(ep // CH, CH)
    src2 = jnp.concatenate([src_p * 2, src_p * 2 + 1]).reshape(2 * ep // CH, CH)
    z128 = jnp.zeros((128, 128), jnp.float32)
    ones128 = jnp.ones((CH, 128), jnp.float32)

    Ws0T, Wn0T, Wsk0T = W_self0.T, W_neigh0.T, W_skip0.T
    Ws1T, Wsk1T = W_self1.T, W_skip1.T
    Wn1AT, Wn1BT = W_neigh1[:, :d_in].T, W_neigh1[:, d_in:].T
    Ws2T = W_self2.T
    Wn2AT, Wn2BT = W_neigh2[:, :d_in].T, W_neigh2[:, d_in:].T
    b0r, b1r, b2r = b0.reshape(1, -1), b1.reshape(1, -1), b2.reshape(1, -1)
    g0r, g1r, g2r = ln_g0.reshape(1, -1), ln_g1.reshape(1, -1), ln_g2.reshape(1, -1)
    be0r, be1r, be2r = ln_b0.reshape(1, -1), ln_b1.reshape(1, -1), ln_b2.reshape(1, -1)
    a0r, a1r, a2r = alpha0.reshape(1, 1), alpha1.reshape(1, 1), alpha2.reshape(1, 1)

    degp = _sc_deg(dst2, z128, ones128, n, ep)
    dg0, dg1 = degp[:n], degp[nsk:nsk + n]
    aggp = _sc_agg(x, src1, dst2, z128, n, ep, split=True)
    h1, in2 = _t1(x, aggp[:n], aggp[nsk:nsk + n], dg0, dg1,
                  Ws0T, Wn0T, Wsk0T, b0r, g0r, be0r, a0r, n, bn)

    agg1 = _sc_agg(in2.reshape(2 * n, d_in), src2, dst2, z128, n, ep, split=False)
    in3 = _t2(x, in2, agg1[:n], agg1[nsk:nsk + n], dg0, dg1, h1,
              Ws1T, Wn1AT, Wn1BT, Wsk1T, b1r, g1r, be1r, a1r, n, bn)

    agg2 = _sc_agg(in3.reshape(2 * n, d_in), src2, dst2, z128, n, ep, split=False)
    ret = _t3(in3, agg2[:n], agg2[nsk:nsk + n], dg0, dg1,
              Ws2T, Wn2AT, Wn2BT, b2r, g2r, be2r, a2r, n, bn)
    return ret


# speedup vs baseline: 1.0202x; 1.0202x over previous
import functools

import jax
import jax.numpy as jnp
from jax import lax
from jax.experimental import pallas as pl
from jax.experimental.pallas import tpu as pltpu
from jax.experimental.pallas import tpu_sc as plsc

NC = 2
NS = 16
CH = 128


def _sc_agg(table, srci, dst2, z128, n, ep, split):
    ept = ep // (NC * NS) if split else ep // NS
    ncz = ept // CH
    gr = NS * 8
    n_sink = ((n + gr - 1) // gr) * gr
    rpt = n_sink // NS

    mesh = plsc.VectorSubcoreMesh(core_axis_name="c", subcore_axis_name="s",
                                  num_cores=NC, num_subcores=NS)
    ib = 16
    assert ncz % ib == 0

    @functools.partial(
        pl.kernel, mesh=mesh,
        out_type=jax.ShapeDtypeStruct((2 * n_sink, 128), jnp.float32),
        scratch_types=[
            pltpu.VMEM_SHARED((n_sink, 128), jnp.float32),
            pltpu.VMEM((ib, CH), jnp.int32),
            pltpu.VMEM((ib, CH), jnp.int32),
            pltpu.VMEM((CH, 128), jnp.float32),
            pltpu.VMEM((CH, 128), jnp.float32),
            pltpu.SemaphoreType.DMA,
            pltpu.SemaphoreType.DMA,
            pltpu.SemaphoreType.DMA,
        ],
    )
    def k(tab_hbm, src_hbm, dst_hbm, z128_hbm, agg_out,
          sp_agg, vm_src, vm_dst, vm_rows0, vm_rows1, sem0, sem1, isem):
        c = lax.axis_index("c")
        s = lax.axis_index("s")
        if split:
            sbase = pl.multiple_of((c * NS + s) * ncz, ncz)
            rbase = sbase
        else:
            sbase = pl.multiple_of(c * (ep // CH) + s * ncz, ncz)
            rbase = pl.multiple_of(s * ncz, ncz)
        pltpu.sync_copy(z128_hbm, vm_rows0)
        zb = pl.multiple_of(s * rpt, 8)
        done = 0
        while done < rpt:
            step = min(128, rpt - done)
            pltpu.sync_copy(vm_rows0.at[pl.ds(0, step)],
                            sp_agg.at[pl.ds(zb + done, step)])
            done += step
        plsc.subcore_barrier()

        bufs = (vm_rows0, vm_rows1)
        sems = (sem0, sem1)

        def body(jo, carry):
            i0 = pltpu.async_copy(
                src_hbm.at[pl.ds(pl.multiple_of(sbase + jo * ib, ib), ib)],
                vm_src, isem)
            i1 = pltpu.async_copy(
                dst_hbm.at[pl.ds(pl.multiple_of(rbase + jo * ib, ib), ib)],
                vm_dst, isem)
            i0.wait()
            i1.wait()
            gd = [None, None]
            gd[0] = pltpu.async_copy(tab_hbm.at[vm_src.at[0]], bufs[0], sems[0])
            for b in range(ib):
                if b + 1 < ib:
                    gd[(b + 1) % 2] = pltpu.async_copy(
                        tab_hbm.at[vm_src.at[b + 1]], bufs[(b + 1) % 2],
                        sems[(b + 1) % 2])
                gd[b % 2].wait()
                pltpu.sync_copy(bufs[b % 2], sp_agg.at[vm_dst.at[b]], add=True)
            return carry

        lax.fori_loop(0, ncz // ib, body, 0)
        plsc.subcore_barrier()

        ob = pl.multiple_of(s * rpt, 8)
        obo = pl.multiple_of(c * n_sink + s * rpt, 8)
        done = 0
        while done < rpt:
            step = min(128, rpt - done)
            pltpu.sync_copy(sp_agg.at[pl.ds(ob + done, step)], vm_rows0.at[pl.ds(0, step)])
            pltpu.sync_copy(vm_rows0.at[pl.ds(0, step)],
                            agg_out.at[pl.ds(pl.multiple_of(obo + done, 8), step)])
            done += step

    return k(table, srci, dst2, z128)


def _sc_deg(dst2, z128, ones128, n, ep):
    ept = ep // (NC * NS)
    ncz = ept // CH
    gr = NS * 8
    n_sink = ((n + gr - 1) // gr) * gr
    rpt = n_sink // NS

    mesh = plsc.VectorSubcoreMesh(core_axis_name="c", subcore_axis_name="s",
                                  num_cores=NC, num_subcores=NS)
    ib = 8
    assert ncz % ib == 0

    @functools.partial(
        pl.kernel, mesh=mesh,
        out_type=jax.ShapeDtypeStruct((2 * n_sink, 128), jnp.float32),
        scratch_types=[
            pltpu.VMEM_SHARED((n_sink, 128), jnp.float32),
            pltpu.VMEM((ib, CH), jnp.int32),
            pltpu.VMEM((128, 128), jnp.float32),
            pltpu.VMEM((CH, 128), jnp.float32),
        ],
    )
    def k(dst_hbm, z128_hbm, ones_hbm, deg_out, sp_deg, vm_dst, vm_z, vm_ones):
        c = lax.axis_index("c")
        s = lax.axis_index("s")
        rbase = pl.multiple_of((c * NS + s) * ncz, ncz)
        pltpu.sync_copy(z128_hbm, vm_z)
        pltpu.sync_copy(ones_hbm, vm_ones)
        zb = pl.multiple_of(s * rpt, 8)
        done = 0
        while done < rpt:
            step = min(128, rpt - done)
            pltpu.sync_copy(vm_z.at[pl.ds(0, step)],
                            sp_deg.at[pl.ds(zb + done, step)])
            done += step
        plsc.subcore_barrier()

        def body(jo, carry):
            pltpu.sync_copy(
                dst_hbm.at[pl.ds(pl.multiple_of(rbase + jo * ib, ib), ib)], vm_dst)
            for b in range(ib):
                pltpu.sync_copy(vm_ones, sp_deg.at[vm_dst.at[b]], add=True)
            return carry

        lax.fori_loop(0, ncz // ib, body, 0)
        plsc.subcore_barrier()

        ob = pl.multiple_of(s * rpt, 8)
        obo = pl.multiple_of(c * n_sink + s * rpt, 8)
        done = 0
        while done < rpt:
            step = min(128, rpt - done)
            pltpu.sync_copy(sp_deg.at[pl.ds(ob + done, step)], vm_z.at[pl.ds(0, step)])
            pltpu.sync_copy(vm_z.at[pl.ds(0, step)],
                            deg_out.at[pl.ds(pl.multiple_of(obo + done, 8), step)])
            done += step

    return k(dst2, z128, ones128)


def _ln_prelu(pre, g, b, al):
    mu = jnp.mean(pre, axis=-1, keepdims=True)
    var = jnp.mean((pre - mu) ** 2, axis=-1, keepdims=True)
    h = (pre - mu) * jax.lax.rsqrt(var + 1e-5) * g + b
    return jnp.where(h >= 0, h, al * h)


def _dot(a, b):
    return jnp.dot(a, b, preferred_element_type=jnp.float32)


def _row_specs(bn, shapes):
    return [pl.BlockSpec((bn,) + tuple(s[1:]),
                         lambda i, r=len(s) - 1: (i,) + (0,) * r)
            for s in shapes]


def _full_specs(shapes):
    return [pl.BlockSpec(tuple(s), lambda i, r=len(s): (0,) * r)
            for s in shapes]


def _t1(x, a0, a1, dg0, dg1, WsT, WnT, WskT, b, g, be, al, n, bn):
    def body(x_r, a0_r, a1_r, dg0_r, dg1_r, WsT_r, WnT_r, WskT_r,
             b_r, g_r, be_r, al_r, h1_r, in2_r):
        x_ = x_r[...]
        agg = a0_r[...] + a1_r[...]
        d = jnp.maximum(dg0_r[:, 0:1] + dg1_r[:, 0:1], 1.0)
        pre = _dot(x_, WsT_r[...]) + _dot(agg, WnT_r[...]) / d + b_r[...]
        h1 = _ln_prelu(pre, g_r[...], be_r[...], al_r[0, 0])
        h1_r[...] = h1
        in2_r[...] = h1 + _dot(x_, WskT_r[...])

    dh = WsT.shape[1]
    return pl.pallas_call(
        body,
        grid=(n // bn,),
        in_specs=_row_specs(bn, [x.shape, a0.shape, a1.shape, dg0.shape, dg1.shape])
        + _full_specs([WsT.shape, WnT.shape, WskT.shape, b.shape, g.shape,
                       be.shape, al.shape]),
        out_specs=_row_specs(bn, [(n, dh), (n, dh)]),
        out_shape=[jax.ShapeDtypeStruct((n, dh), jnp.float32),
                   jax.ShapeDtypeStruct((n, dh), jnp.float32)],
    )(x, a0, a1, dg0, dg1, WsT, WnT, WskT, b, g, be, al)


def _t2(x, in2, ah0, ah1, dg0, dg1, h1, WsT, WnAT, WnBT, WskT, b, g, be, al,
        n, bn):
    def body(x_r, in2_r, ah0_r, ah1_r, dg0_r, dg1_r, h1_r,
             WsT_r, WnAT_r, WnBT_r, WskT_r, b_r, g_r, be_r, al_r, in3_r):
        d = jnp.maximum(dg0_r[:, 0:1] + dg1_r[:, 0:1], 1.0)
        neigh = (_dot(ah0_r[...], WnAT_r[...]) + _dot(ah1_r[...], WnBT_r[...])) / d
        pre = _dot(in2_r[...], WsT_r[...]) + neigh + b_r[...]
        h2 = _ln_prelu(pre, g_r[...], be_r[...], al_r[0, 0])
        in3_r[...] = h1_r[...] + h2 + _dot(x_r[...], WskT_r[...])

    dh = WsT.shape[1]
    return pl.pallas_call(
        body,
        grid=(n // bn,),
        in_specs=_row_specs(bn, [x.shape, in2.shape, ah0.shape, ah1.shape,
                                 dg0.shape, dg1.shape, h1.shape])
        + _full_specs([WsT.shape, WnAT.shape, WnBT.shape, WskT.shape,
                       b.shape, g.shape, be.shape, al.shape]),
        out_specs=_row_specs(bn, [(n, dh)])[0],
        out_shape=jax.ShapeDtypeStruct((n, dh), jnp.float32),
    )(x, in2, ah0, ah1, dg0, dg1, h1, WsT, WnAT, WnBT, WskT, b, g, be, al)


def _t3(in3, ah0, ah1, dg0, dg1, WsT, WnAT, WnBT, b, g, be, al, n, bn):
    def body(in3_r, ah0_r, ah1_r, dg0_r, dg1_r,
             WsT_r, WnAT_r, WnBT_r, b_r, g_r, be_r, al_r, out_r):
        d = jnp.maximum(dg0_r[:, 0:1] + dg1_r[:, 0:1], 1.0)
        neigh = (_dot(ah0_r[...], WnAT_r[...]) + _dot(ah1_r[...], WnBT_r[...])) / d
        pre = _dot(in3_r[...], WsT_r[...]) + neigh + b_r[...]
        out_r[...] = _ln_prelu(pre, g_r[...], be_r[...], al_r[0, 0])

    dh = WsT.shape[1]
    return pl.pallas_call(
        body,
        grid=(n // bn,),
        in_specs=_row_specs(bn, [in3.shape, ah0.shape, ah1.shape,
                                 dg0.shape, dg1.shape])
        + _full_specs([WsT.shape, WnAT.shape, WnBT.shape, b.shape, g.shape,
                       be.shape, al.shape]),
        out_specs=_row_specs(bn, [(n, dh)])[0],
        out_shape=jax.ShapeDtypeStruct((n, dh), jnp.float32),
    )(in3, ah0, ah1, dg0, dg1, WsT, WnAT, WnBT, b, g, be, al)


def kernel(x, edge_index, W_self0, W_neigh0, b0, W_self1, W_neigh1, b1,
           W_self2, W_neigh2, b2, W_skip0, W_skip1,
           ln_g0, ln_b0, ln_g1, ln_b1, ln_g2, ln_b2,
           alpha0, alpha1, alpha2):
    n, d_in = x.shape
    dh = W_self0.shape[0]
    e = edge_index.shape[1]
    bn = 2000

    gran = NC * NS * CH * 8
    ep = ((e + gran - 1) // gran) * gran
    nsk = ((n + NS * 8 - 1) // (NS * 8)) * (NS * 8)
    pad = ep - e
    src = edge_index[0]
    dst = edge_index[1]
    src_p = jnp.concatenate([src, jnp.zeros((pad,), jnp.int32)])
    dst_p = jnp.concatenate([dst, jnp.full((pad,), n, jnp.int32)])
    dst2 = dst_p.reshape(ep // CH, CH)
    src1 = src_p.reshape(ep // CH, CH)
    src2 = jnp.concatenate([src_p * 2, src_p * 2 + 1]).reshape(2 * ep // CH, CH)
    z128 = jnp.zeros((128, 128), jnp.float32)
    ones128 = jnp.ones((CH, 128), jnp.float32)

    Ws0T, Wn0T, Wsk0T = W_self0.T, W_neigh0.T, W_skip0.T
    Ws1T, Wsk1T = W_self1.T, W_skip1.T
    Wn1AT, Wn1BT = W_neigh1[:, :d_in].T, W_neigh1[:, d_in:].T
    Ws2T = W_self2.T
    Wn2AT, Wn2BT = W_neigh2[:, :d_in].T, W_neigh2[:, d_in:].T
    b0r, b1r, b2r = b0.reshape(1, -1), b1.reshape(1, -1), b2.reshape(1, -1)
    g0r, g1r, g2r = ln_g0.reshape(1, -1), ln_g1.reshape(1, -1), ln_g2.reshape(1, -1)
    be0r, be1r, be2r = ln_b0.reshape(1, -1), ln_b1.reshape(1, -1), ln_b2.reshape(1, -1)
    a0r, a1r, a2r = alpha0.reshape(1, 1), alpha1.reshape(1, 1), alpha2.reshape(1, 1)

    degp = _sc_deg(dst2, z128, ones128, n, ep)
    dg0, dg1 = degp[:n], degp[nsk:nsk + n]
    aggp = _sc_agg(x, src1, dst2, z128, n, ep, split=True)
    h1, in2 = _t1(x, aggp[:n], aggp[nsk:nsk + n], dg0, dg1,
                  Ws0T, Wn0T, Wsk0T, b0r, g0r, be0r, a0r, n, bn)

    agg1 = _sc_agg(in2.reshape(2 * n, d_in), src2, dst2, z128, n, ep, split=False)
    in3 = _t2(x, in2, agg1[:n], agg1[nsk:nsk + n], dg0, dg1, h1,
              Ws1T, Wn1AT, Wn1BT, Wsk1T, b1r, g1r, be1r, a1r, n, bn)

    agg2 = _sc_agg(in3.reshape(2 * n, d_in), src2, dst2, z128, n, ep, split=False)
    ret = _t3(in3, agg2[:n], agg2[nsk:nsk + n], dg0, dg1,
              Ws2T, Wn2AT, Wn2BT, b2r, g2r, be2r, a2r, n, bn)
    return ret
